# baseline (speedup 1.0000x reference)
import functools
import jax
import jax.numpy as jnp
from jax import lax
from jax.experimental import pallas as pl
from jax.experimental.pallas import tpu as pltpu
from jax.experimental.pallas import tpu_sc as plsc

N = 10000
E = 320000
D_IN = 128
D_HID = 256
D_OUT = 128
G = 64
D = 256
DH = 128

N2 = 10240
BR = 640
NBLK = N2 // BR
CH = 80
NCH = E // 16 // CH
RPT = N2 // 16

_mesh = plsc.VectorSubcoreMesh(core_axis_name="c", subcore_axis_name="s")


def _deg_body(dst3, deg0, deg1, idx_v, ones_v, zbuf_v, acc):
    c = lax.axis_index("c")
    s = lax.axis_index("s")
    w = s * 2 + c

    for k in range(CH // 16):
        ones_v[pl.ds(k * 16, 16)] = jnp.full((16,), 1.0, jnp.float32)

    def zfill(i, _):
        zbuf_v[pl.ds(i * 16, 16)] = jnp.zeros((16,), jnp.float32)
        return 0
    lax.fori_loop(0, RPT // 16, zfill, 0)

    pltpu.sync_copy(zbuf_v, acc.at[pl.ds(s * RPT, RPT)])
    plsc.subcore_barrier()

    pltpu.sync_copy(dst3.at[w], idx_v)

    def step(j, _):
        pltpu.sync_copy(ones_v, acc.at[idx_v.at[j]], add=True)
        return 0
    lax.fori_loop(0, E // 32 // CH, step, 0)

    plsc.subcore_barrier()
    sl = pl.ds(s * RPT, RPT)

    @pl.when(c == 0)
    def _():
        pltpu.sync_copy(acc.at[sl], deg0.at[sl])

    @pl.when(c == 1)
    def _():
        pltpu.sync_copy(acc.at[sl], deg1.at[sl])


_deg_call = pl.kernel(
    _deg_body,
    mesh=_mesh,
    out_type=[jax.ShapeDtypeStruct((N2,), jnp.float32),
              jax.ShapeDtypeStruct((N2,), jnp.float32)],
    scratch_types=[
        pltpu.VMEM((E // 32 // CH, CH), jnp.int32),
        pltpu.VMEM((CH,), jnp.float32),
        pltpu.VMEM((RPT,), jnp.float32),
        pltpu.VMEM_SHARED((N2,), jnp.float32),
    ],
)


GRP = 20
NGRP = 13
EPT = NGRP * GRP * CH
ACC_ROWS = 10112
APT = ACC_ROWS // 16


def _pipe_chunks(src3, dst3, h_ref, acc, base, ngrp, grp,
                 idxs_v, idxd_v, rows_a, rows_b,
                 sem_a, sem_b, sem_c, sem_d):
    def group(g, _):
        pltpu.sync_copy(src3.at[base + g], idxs_v)
        pltpu.sync_copy(dst3.at[base + g], idxd_v)
        gsem = [sem_a, sem_b]
        ssem = [sem_c, sem_d]
        bufs = [rows_a, rows_b]
        cps = [None] * grp
        sps = [None] * grp
        cps[0] = pltpu.async_copy(h_ref.at[idxs_v.at[0]], rows_a, sem_a)
        for j in range(grp):
            p = j % 2
            cps[j].wait()
            if j + 1 < grp:
                if j >= 1:
                    sps[j - 1].wait()
                cps[j + 1] = pltpu.async_copy(
                    h_ref.at[idxs_v.at[j + 1]], bufs[1 - p], gsem[1 - p])
            sps[j] = pltpu.async_copy(
                bufs[p], acc.at[idxd_v.at[j]], ssem[p], add=True)
        sps[grp - 2].wait()
        sps[grp - 1].wait()
        return 0
    lax.fori_loop(0, ngrp, group, 0)


def _agg_body(h0, h1, src3, dst3, a0, a1,
              idxs_v, idxd_v, rows_a, rows_b, acc,
              sem_a, sem_b, sem_c, sem_d):
    c = lax.axis_index("c")
    s = lax.axis_index("s")

    def zfill(r, _):
        for k in range(DH // 16):
            rows_a[r, pl.ds(k * 16, 16)] = jnp.zeros((16,), jnp.float32)
        return 0
    lax.fori_loop(0, CH, zfill, 0)

    for k in range(7):
        pltpu.sync_copy(rows_a, acc.at[pl.ds(s * APT + k * CH, CH)])
    pltpu.sync_copy(rows_a.at[pl.ds(0, APT - 7 * CH)],
                    acc.at[pl.ds(s * APT + 7 * CH, APT - 7 * CH)])
    plsc.subcore_barrier()

    def run(h_ref, a_ref):
        _pipe_chunks(src3, dst3, h_ref, acc, s * NGRP, NGRP, GRP,
                     idxs_v, idxd_v, rows_a, rows_b,
                     sem_a, sem_b, sem_c, sem_d)
        plsc.subcore_barrier()
        sl = pl.ds(s * APT, APT)
        pltpu.sync_copy(acc.at[sl], a_ref.at[sl])

    @pl.when(c == 0)
    def _():
        run(h0, a0)

    @pl.when(c == 1)
    def _():
        run(h1, a1)


_vh = jax.ShapeDtypeStruct((N2, DH), jnp.float32)
_agg_call = pl.kernel(
    _agg_body,
    mesh=_mesh,
    out_type=[_vh, _vh],
    scratch_types=[
        pltpu.VMEM((GRP, CH), jnp.int32),
        pltpu.VMEM((GRP, CH), jnp.int32),
        pltpu.VMEM((CH, DH), jnp.float32),
        pltpu.VMEM((CH, DH), jnp.float32),
        pltpu.VMEM_SHARED((ACC_ROWS, DH), jnp.float32),
        pltpu.SemaphoreType.DMA,
        pltpu.SemaphoreType.DMA,
        pltpu.SemaphoreType.DMA,
        pltpu.SemaphoreType.DMA,
    ],
)


GRP3 = 10
NGRP3 = 13
EPT3 = NGRP3 * GRP3 * CH


def _agg3_body(h, src3, dst3, a0, a1,
               idxs_v, idxd_v, rows_a, rows_b, acc,
               sem_a, sem_b, sem_c, sem_d):
    c = lax.axis_index("c")
    s = lax.axis_index("s")
    w = c * 16 + s

    def zfill(r, _):
        for k in range(DH // 16):
            rows_a[r, pl.ds(k * 16, 16)] = jnp.zeros((16,), jnp.float32)
        return 0
    lax.fori_loop(0, CH, zfill, 0)

    for k in range(7):
        pltpu.sync_copy(rows_a, acc.at[pl.ds(s * APT + k * CH, CH)])
    pltpu.sync_copy(rows_a.at[pl.ds(0, APT - 7 * CH)],
                    acc.at[pl.ds(s * APT + 7 * CH, APT - 7 * CH)])
    plsc.subcore_barrier()

    _pipe_chunks(src3, dst3, h, acc, w * NGRP3, NGRP3, GRP3,
                 idxs_v, idxd_v, rows_a, rows_b,
                 sem_a, sem_b, sem_c, sem_d)

    plsc.subcore_barrier()
    sl = pl.ds(s * APT, APT)

    @pl.when(c == 0)
    def _():
        pltpu.sync_copy(acc.at[sl], a0.at[sl])

    @pl.when(c == 1)
    def _():
        pltpu.sync_copy(acc.at[sl], a1.at[sl])


_agg3_call = pl.kernel(
    _agg3_body,
    mesh=_mesh,
    out_type=[_vh, _vh],
    scratch_types=[
        pltpu.VMEM((GRP3, CH), jnp.int32),
        pltpu.VMEM((GRP3, CH), jnp.int32),
        pltpu.VMEM((CH, DH), jnp.float32),
        pltpu.VMEM((CH, DH), jnp.float32),
        pltpu.VMEM_SHARED((ACC_ROWS, DH), jnp.float32),
        pltpu.SemaphoreType.DMA,
        pltpu.SemaphoreType.DMA,
        pltpu.SemaphoreType.DMA,
        pltpu.SemaphoreType.DMA,
    ],
)


def _pre_body(x_ref, w_ref, dv_ref, h0_ref, h1_ref):
    h = jnp.dot(x_ref[...], w_ref[...], preferred_element_type=jnp.float32)
    h = h * dv_ref[:, 0:1]
    h0_ref[...] = h[:, :DH]
    h1_ref[...] = h[:, DH:]


def _dense_pre(x, w, dinvb):
    din = w.shape[0]
    return pl.pallas_call(
        _pre_body,
        grid=(NBLK,),
        in_specs=[
            pl.BlockSpec((BR, din), lambda i: (i, 0)),
            pl.BlockSpec((din, D), lambda i: (0, 0)),
            pl.BlockSpec((BR, 128), lambda i: (i, 0)),
        ],
        out_specs=[pl.BlockSpec((BR, DH), lambda i: (i, 0))] * 2,
        out_shape=[_vh, _vh],
    )(x, w, dinvb)


def _postpre_body(nout, a0, a1, h0, h1, dv_ref, b_ref, g_ref, bt_ref,
                  w_ref, *o_refs):
    t = jnp.concatenate([a0[...] + h0[...], a1[...] + h1[...]], axis=-1)
    t = t * dv_ref[:, 0:1] + b_ref[...]
    r = jnp.maximum(t, 0.0)
    mu = jnp.mean(r, axis=-1, keepdims=True)
    var = jnp.mean(jnp.square(r - mu), axis=-1, keepdims=True)
    ln = (r - mu) * lax.rsqrt(var + 1e-5) * g_ref[...] + bt_ref[...]
    h = jnp.dot(ln, w_ref[...], preferred_element_type=jnp.float32)
    h = h * dv_ref[:, 0:1]
    if nout == 2:
        o_refs[0][...] = h[:, :DH]
        o_refs[1][...] = h[:, DH:]
    else:
        o_refs[0][...] = h


def _dense_postpre(a0, a1, h0, h1, dinvb, b, g, bt, wn):
    dn = wn.shape[1]
    nout = dn // DH
    vspec = pl.BlockSpec((BR, DH), lambda i: (i, 0))
    rspec = pl.BlockSpec((1, D), lambda i: (0, 0))
    return pl.pallas_call(
        functools.partial(_postpre_body, nout),
        grid=(NBLK,),
        in_specs=[vspec, vspec, vspec, vspec,
                  pl.BlockSpec((BR, 128), lambda i: (i, 0)),
                  rspec, rspec, rspec,
                  pl.BlockSpec((D, dn), lambda i: (0, 0))],
        out_specs=[pl.BlockSpec((BR, DH), lambda i: (i, 0))] * nout,
        out_shape=[_vh] * nout,
    )(a0, a1, h0, h1, dinvb, b.reshape(1, D), g.reshape(1, D),
      bt.reshape(1, D), wn)


def _post3_body(a0, a1, h_ref, dv_ref, b_ref, bt_ref, o_ref, z_ref,
                sums, cnt):
    i = pl.program_id(0)
    t = a0[...] + a1[...] + h_ref[...]
    t = t * dv_ref[:, 0:1] + b_ref[...]
    o_ref[...] = t

    @pl.when(i == 0)
    def _():
        sums[...] = jnp.zeros_like(sums)
        cnt[...] = jnp.zeros_like(cnt)

    bb = bt_ref[0, 0, :]
    oh = (bb[:, None] == lax.broadcasted_iota(jnp.int32, (BR, G), 1)
          ).astype(jnp.float32)
    valid = jnp.sum(oh, axis=1, keepdims=True)
    tc = jnp.where(valid > 0.0, t, 0.0)
    sums[...] += lax.dot_general(oh, tc, (((0,), (0,)), ((), ())),
                                 preferred_element_type=jnp.float32)
    cnt[...] += jnp.broadcast_to(jnp.sum(oh, axis=0)[:, None], cnt.shape)
    z_ref[...] = sums[...] / jnp.maximum(cnt[...], 1.0)


def _post3pool(a0, a1, h, dinvb, b, batch3):
    vspec = pl.BlockSpec((BR, DH), lambda i: (i, 0))
    return pl.pallas_call(
        _post3_body,
        grid=(NBLK,),
        in_specs=[vspec, vspec, vspec,
                  pl.BlockSpec((BR, 128), lambda i: (i, 0)),
                  pl.BlockSpec((1, DH), lambda i: (0, 0)),
                  pl.BlockSpec((1, 1, BR), lambda i: (i, 0, 0))],
        out_specs=[vspec, pl.BlockSpec((G, D_OUT), lambda i: (0, 0))],
        out_shape=[_vh, jax.ShapeDtypeStruct((G, D_OUT), jnp.float32)],
        scratch_shapes=[pltpu.VMEM((G, D_OUT), jnp.float32),
                        pltpu.VMEM((G, D_OUT), jnp.float32)],
    )(a0, a1, h, dinvb, b.reshape(1, DH), batch3)


def kernel(x, W1, b1, g1, bt1, W2, b2, g2, bt2, W3, b3, edge_index, batch):
    f32 = jnp.float32
    src = edge_index[0]
    dst = edge_index[1]
    npad = 16 * EPT - E
    trash = N + (jnp.arange(npad, dtype=jnp.int32) % (ACC_ROWS - N))
    src_p = jnp.concatenate([src, jnp.zeros((npad,), jnp.int32)])
    dst_p = jnp.concatenate([dst, trash])
    src3 = src_p.reshape(16 * NGRP, GRP, CH)
    dst3 = dst_p.reshape(16 * NGRP, GRP, CH)
    dst3d = dst.reshape(32, E // 32 // CH, CH)

    deg0, deg1 = _deg_call(dst3d)
    dinv = lax.rsqrt(deg0 + deg1 + 1.0)
    dinvb = jnp.broadcast_to(dinv[:, None], (N2, 128))

    x2 = jnp.zeros((N2, D_IN), f32).at[:N].set(x)
    batch2 = jnp.full((N2,), G, jnp.int32).at[:N].set(batch)
    batch3 = batch2.reshape(NBLK, 1, BR)
    npad3 = 32 * EPT3 - E
    trash3 = N + (jnp.arange(npad3, dtype=jnp.int32) % (ACC_ROWS - N))
    src3e = jnp.concatenate([src, jnp.zeros((npad3,), jnp.int32)]
                            ).reshape(32 * NGRP3, GRP3, CH)
    dst3e = jnp.concatenate([dst, trash3]
                            ).reshape(32 * NGRP3, GRP3, CH)

    h0, h1 = _dense_pre(x2, W1, dinvb)
    a0, a1 = _agg_call(h0, h1, src3, dst3)
    h0, h1 = _dense_postpre(a0, a1, h0, h1, dinvb, b1, g1, bt1, W2)
    a0, a1 = _agg_call(h0, h1, src3, dst3)
    (h3,) = _dense_postpre(a0, a1, h0, h1, dinvb, b2, g2, bt2, W3)
    a0, a1 = _agg3_call(h3, src3e, dst3e)
    hf, z = _post3pool(a0, a1, h3, dinvb, b3, batch3)
    return (hf[:N], z)


# speedup vs baseline: 2.9167x; 2.9167x over previous
import functools
import jax
import jax.numpy as jnp
from jax import lax
from jax.experimental import pallas as pl
from jax.experimental.pallas import tpu as pltpu
from jax.experimental.pallas import tpu_sc as plsc

N = 10000
E = 320000
D_IN = 128
D_HID = 256
D_OUT = 128
G = 64
D = 256
DH = 128

N2 = 10240
BR = 640
NBLK = N2 // BR
CH = 80
NCH = E // 16 // CH
RPT = N2 // 16

_mesh = plsc.VectorSubcoreMesh(core_axis_name="c", subcore_axis_name="s")


def _deg_body(dst3, deg0, deg1, idx_v, ones_v, zbuf_v, acc):
    c = lax.axis_index("c")
    s = lax.axis_index("s")
    w = s * 2 + c

    for k in range(CH // 16):
        ones_v[pl.ds(k * 16, 16)] = jnp.full((16,), 1.0, jnp.float32)

    def zfill(i, _):
        zbuf_v[pl.ds(i * 16, 16)] = jnp.zeros((16,), jnp.float32)
        return 0
    lax.fori_loop(0, RPT // 16, zfill, 0)

    pltpu.sync_copy(zbuf_v, acc.at[pl.ds(s * RPT, RPT)])
    plsc.subcore_barrier()

    pltpu.sync_copy(dst3.at[w], idx_v)

    def step(j, _):
        pltpu.sync_copy(ones_v, acc.at[idx_v.at[j]], add=True)
        return 0
    lax.fori_loop(0, E // 32 // CH, step, 0)

    plsc.subcore_barrier()
    sl = pl.ds(s * RPT, RPT)

    @pl.when(c == 0)
    def _():
        pltpu.sync_copy(acc.at[sl], deg0.at[sl])

    @pl.when(c == 1)
    def _():
        pltpu.sync_copy(acc.at[sl], deg1.at[sl])


_deg_call = pl.kernel(
    _deg_body,
    mesh=_mesh,
    out_type=[jax.ShapeDtypeStruct((N2,), jnp.float32),
              jax.ShapeDtypeStruct((N2,), jnp.float32)],
    scratch_types=[
        pltpu.VMEM((E // 32 // CH, CH), jnp.int32),
        pltpu.VMEM((CH,), jnp.float32),
        pltpu.VMEM((RPT,), jnp.float32),
        pltpu.VMEM_SHARED((N2,), jnp.float32),
    ],
)


GRP = 10
NGRP = NCH // GRP
ACC_ROWS = 10112
APT = ACC_ROWS // 16


def _pipe_chunks(src3, dst3, h_ref, acc, base, ngrp, grp,
                 idxs_v, idxd_v, rows_a, rows_b,
                 sem_a, sem_b, sem_c, sem_d):
    def group(g, _):
        pltpu.sync_copy(src3.at[base + g], idxs_v)
        pltpu.sync_copy(dst3.at[base + g], idxd_v)
        gsem = [sem_a, sem_b]
        ssem = [sem_c, sem_d]
        bufs = [rows_a, rows_b]
        cps = [None] * grp
        sps = [None] * grp
        cps[0] = pltpu.async_copy(h_ref.at[idxs_v.at[0]], rows_a, sem_a)
        for j in range(grp):
            p = j % 2
            cps[j].wait()
            if j + 1 < grp:
                if j >= 1:
                    sps[j - 1].wait()
                cps[j + 1] = pltpu.async_copy(
                    h_ref.at[idxs_v.at[j + 1]], bufs[1 - p], gsem[1 - p])
            sps[j] = pltpu.async_copy(
                bufs[p], acc.at[idxd_v.at[j]], ssem[p], add=True)
        sps[grp - 2].wait()
        sps[grp - 1].wait()
        return 0
    lax.fori_loop(0, ngrp, group, 0)


def _agg_body(h0, h1, src3, dst3, a0, a1,
              idxs_v, idxd_v, rows_a, rows_b, acc,
              sem_a, sem_b, sem_c, sem_d):
    c = lax.axis_index("c")
    s = lax.axis_index("s")

    def zfill(r, _):
        for k in range(DH // 16):
            rows_a[r, pl.ds(k * 16, 16)] = jnp.zeros((16,), jnp.float32)
        return 0
    lax.fori_loop(0, CH, zfill, 0)

    for k in range(7):
        pltpu.sync_copy(rows_a, acc.at[pl.ds(s * APT + k * CH, CH)])
    pltpu.sync_copy(rows_a.at[pl.ds(0, APT - 7 * CH)],
                    acc.at[pl.ds(s * APT + 7 * CH, APT - 7 * CH)])
    plsc.subcore_barrier()

    def run(h_ref, a_ref):
        _pipe_chunks(src3, dst3, h_ref, acc, s * NGRP, NGRP, GRP,
                     idxs_v, idxd_v, rows_a, rows_b,
                     sem_a, sem_b, sem_c, sem_d)
        plsc.subcore_barrier()
        sl = pl.ds(s * APT, APT)
        pltpu.sync_copy(acc.at[sl], a_ref.at[sl])

    @pl.when(c == 0)
    def _():
        run(h0, a0)

    @pl.when(c == 1)
    def _():
        run(h1, a1)


_vh = jax.ShapeDtypeStruct((N2, DH), jnp.float32)
_agg_call = pl.kernel(
    _agg_body,
    mesh=_mesh,
    out_type=[_vh, _vh],
    scratch_types=[
        pltpu.VMEM((GRP, CH), jnp.int32),
        pltpu.VMEM((GRP, CH), jnp.int32),
        pltpu.VMEM((CH, DH), jnp.float32),
        pltpu.VMEM((CH, DH), jnp.float32),
        pltpu.VMEM_SHARED((ACC_ROWS, DH), jnp.float32),
        pltpu.SemaphoreType.DMA,
        pltpu.SemaphoreType.DMA,
        pltpu.SemaphoreType.DMA,
        pltpu.SemaphoreType.DMA,
    ],
)


GRP3 = 5
NGRP3 = (E // 32 // CH) // GRP3


def _agg3_body(h, src3, dst3, a0, a1,
               idxs_v, idxd_v, rows_a, rows_b, acc,
               sem_a, sem_b, sem_c, sem_d):
    c = lax.axis_index("c")
    s = lax.axis_index("s")
    w = c * 16 + s

    def zfill(r, _):
        for k in range(DH // 16):
            rows_a[r, pl.ds(k * 16, 16)] = jnp.zeros((16,), jnp.float32)
        return 0
    lax.fori_loop(0, CH, zfill, 0)

    for k in range(7):
        pltpu.sync_copy(rows_a, acc.at[pl.ds(s * APT + k * CH, CH)])
    pltpu.sync_copy(rows_a.at[pl.ds(0, APT - 7 * CH)],
                    acc.at[pl.ds(s * APT + 7 * CH, APT - 7 * CH)])
    plsc.subcore_barrier()

    _pipe_chunks(src3, dst3, h, acc, w * NGRP3, NGRP3, GRP3,
                 idxs_v, idxd_v, rows_a, rows_b,
                 sem_a, sem_b, sem_c, sem_d)

    plsc.subcore_barrier()
    sl = pl.ds(s * APT, APT)

    @pl.when(c == 0)
    def _():
        pltpu.sync_copy(acc.at[sl], a0.at[sl])

    @pl.when(c == 1)
    def _():
        pltpu.sync_copy(acc.at[sl], a1.at[sl])


_agg3_call = pl.kernel(
    _agg3_body,
    mesh=_mesh,
    out_type=[_vh, _vh],
    scratch_types=[
        pltpu.VMEM((GRP3, CH), jnp.int32),
        pltpu.VMEM((GRP3, CH), jnp.int32),
        pltpu.VMEM((CH, DH), jnp.float32),
        pltpu.VMEM((CH, DH), jnp.float32),
        pltpu.VMEM_SHARED((ACC_ROWS, DH), jnp.float32),
        pltpu.SemaphoreType.DMA,
        pltpu.SemaphoreType.DMA,
        pltpu.SemaphoreType.DMA,
        pltpu.SemaphoreType.DMA,
    ],
)


def _pre_body(x_ref, w_ref, dv_ref, h0_ref, h1_ref):
    h = jnp.dot(x_ref[...], w_ref[...], preferred_element_type=jnp.float32)
    h = h * dv_ref[:, 0:1]
    h0_ref[...] = h[:, :DH]
    h1_ref[...] = h[:, DH:]


def _dense_pre(x, w, dinvb):
    din = w.shape[0]
    return pl.pallas_call(
        _pre_body,
        grid=(NBLK,),
        in_specs=[
            pl.BlockSpec((BR, din), lambda i: (i, 0)),
            pl.BlockSpec((din, D), lambda i: (0, 0)),
            pl.BlockSpec((BR, 128), lambda i: (i, 0)),
        ],
        out_specs=[pl.BlockSpec((BR, DH), lambda i: (i, 0))] * 2,
        out_shape=[_vh, _vh],
    )(x, w, dinvb)


def _postpre_body(nout, a0, a1, h0, h1, dv_ref, b_ref, g_ref, bt_ref,
                  w_ref, *o_refs):
    t = jnp.concatenate([a0[...] + h0[...], a1[...] + h1[...]], axis=-1)
    t = t * dv_ref[:, 0:1] + b_ref[...]
    r = jnp.maximum(t, 0.0)
    mu = jnp.mean(r, axis=-1, keepdims=True)
    var = jnp.mean(jnp.square(r - mu), axis=-1, keepdims=True)
    ln = (r - mu) * lax.rsqrt(var + 1e-5) * g_ref[...] + bt_ref[...]
    h = jnp.dot(ln, w_ref[...], preferred_element_type=jnp.float32)
    h = h * dv_ref[:, 0:1]
    if nout == 2:
        o_refs[0][...] = h[:, :DH]
        o_refs[1][...] = h[:, DH:]
    else:
        o_refs[0][...] = h


def _dense_postpre(a0, a1, h0, h1, dinvb, b, g, bt, wn):
    dn = wn.shape[1]
    nout = dn // DH
    vspec = pl.BlockSpec((BR, DH), lambda i: (i, 0))
    rspec = pl.BlockSpec((1, D), lambda i: (0, 0))
    return pl.pallas_call(
        functools.partial(_postpre_body, nout),
        grid=(NBLK,),
        in_specs=[vspec, vspec, vspec, vspec,
                  pl.BlockSpec((BR, 128), lambda i: (i, 0)),
                  rspec, rspec, rspec,
                  pl.BlockSpec((D, dn), lambda i: (0, 0))],
        out_specs=[pl.BlockSpec((BR, DH), lambda i: (i, 0))] * nout,
        out_shape=[_vh] * nout,
    )(a0, a1, h0, h1, dinvb, b.reshape(1, D), g.reshape(1, D),
      bt.reshape(1, D), wn)


def _post3_body(a0, a1, h_ref, dv_ref, b_ref, bt_ref, o_ref, z_ref,
                sums, cnt):
    i = pl.program_id(0)
    t = a0[...] + a1[...] + h_ref[...]
    t = t * dv_ref[:, 0:1] + b_ref[...]
    o_ref[...] = t

    @pl.when(i == 0)
    def _():
        sums[...] = jnp.zeros_like(sums)
        cnt[...] = jnp.zeros_like(cnt)

    bb = bt_ref[0, 0, :]
    oh = (bb[:, None] == lax.broadcasted_iota(jnp.int32, (BR, G), 1)
          ).astype(jnp.float32)
    valid = jnp.sum(oh, axis=1, keepdims=True)
    tc = jnp.where(valid > 0.0, t, 0.0)
    sums[...] += lax.dot_general(oh, tc, (((0,), (0,)), ((), ())),
                                 preferred_element_type=jnp.float32)
    cnt[...] += jnp.broadcast_to(jnp.sum(oh, axis=0)[:, None], cnt.shape)
    z_ref[...] = sums[...] / jnp.maximum(cnt[...], 1.0)


def _post3pool(a0, a1, h, dinvb, b, batch3):
    vspec = pl.BlockSpec((BR, DH), lambda i: (i, 0))
    return pl.pallas_call(
        _post3_body,
        grid=(NBLK,),
        in_specs=[vspec, vspec, vspec,
                  pl.BlockSpec((BR, 128), lambda i: (i, 0)),
                  pl.BlockSpec((1, DH), lambda i: (0, 0)),
                  pl.BlockSpec((1, 1, BR), lambda i: (i, 0, 0))],
        out_specs=[vspec, pl.BlockSpec((G, D_OUT), lambda i: (0, 0))],
        out_shape=[_vh, jax.ShapeDtypeStruct((G, D_OUT), jnp.float32)],
        scratch_shapes=[pltpu.VMEM((G, D_OUT), jnp.float32),
                        pltpu.VMEM((G, D_OUT), jnp.float32)],
    )(a0, a1, h, dinvb, b.reshape(1, DH), batch3)


def kernel(x, W1, b1, g1, bt1, W2, b2, g2, bt2, W3, b3, edge_index, batch):
    f32 = jnp.float32
    src = edge_index[0]
    dst = edge_index[1]
    src3 = src.reshape(16 * NGRP, GRP, CH)
    dst3 = dst.reshape(16 * NGRP, GRP, CH)
    dst3d = dst.reshape(32, E // 32 // CH, CH)

    deg0, deg1 = _deg_call(dst3d)
    dinv = lax.rsqrt(deg0 + deg1 + 1.0)
    dinvb = jnp.broadcast_to(dinv[:, None], (N2, 128))

    x2 = jnp.zeros((N2, D_IN), f32).at[:N].set(x)
    batch2 = jnp.full((N2,), G, jnp.int32).at[:N].set(batch)
    batch3 = batch2.reshape(NBLK, 1, BR)
    src3e = src.reshape(32 * NGRP3, GRP3, CH)
    dst3e = dst.reshape(32 * NGRP3, GRP3, CH)

    h0, h1 = _dense_pre(x2, W1, dinvb)
    a0, a1 = _agg_call(h0, h1, src3, dst3)
    h0, h1 = _dense_postpre(a0, a1, h0, h1, dinvb, b1, g1, bt1, W2)
    a0, a1 = _agg_call(h0, h1, src3, dst3)
    (h3,) = _dense_postpre(a0, a1, h0, h1, dinvb, b2, g2, bt2, W3)
    a0, a1 = _agg3_call(h3, src3e, dst3e)
    hf, z = _post3pool(a0, a1, h3, dinvb, b3, batch3)
    return (hf[:N], z)
